# baseline (speedup 1.0000x reference)
import functools

import jax
import jax.numpy as jnp
import numpy as np
from jax import lax
from jax.experimental import pallas as pl
from jax.experimental.pallas import tpu as pltpu
from jax.experimental.pallas import tpu_sc as plsc

_N = 50000
_E = 800000
_D = 100
_H = 32
_C = 2

_NC = 2
_NS = 16
_NWK = _NC * _NS
_W = 128
_KB = 25
_NBLKE = _E // (_W * _KB)
_NP = 50048
_CHUNK = _NP // _NS
_DEGW = 32
_NR = 5
_LAG = 3

_mesh = plsc.VectorSubcoreMesh(core_axis_name="c", subcore_axis_name="s")


def _worker_blocks(wid):
    return (_NBLKE - wid + _NWK - 1) // _NWK


def _ring_pass(e3_hbm, srcb, dstb, wid, table_ref, acc_sh, bufs, gsems, ssems):
    def body(i, carry):
        blk = wid + i * _NWK
        pltpu.sync_copy(e3_hbm.at[0, pl.ds(blk * _KB, _KB)], srcb)
        pltpu.sync_copy(e3_hbm.at[1, pl.ds(blk * _KB, _KB)], dstb)
        gcp = [None] * _KB
        scp = [None] * _KB
        for j in range(_KB):
            b = j % _NR
            if j >= _NR:
                scp[j - _NR].wait()
            gcp[j] = pltpu.async_copy(table_ref.at[srcb.at[j]], bufs[b],
                                      gsems[b])
            if j >= _LAG:
                p = j - _LAG
                gcp[p].wait()
                scp[p] = pltpu.async_copy(bufs[p % _NR],
                                          acc_sh.at[dstb.at[p]],
                                          ssems[p % _NR], add=True)
        for p in range(_KB - _LAG, _KB):
            gcp[p].wait()
            scp[p] = pltpu.async_copy(bufs[p % _NR], acc_sh.at[dstb.at[p]],
                                      ssems[p % _NR], add=True)
        for p in range(_KB - _NR, _KB):
            scp[p].wait()
        return carry

    lax.fori_loop(0, _worker_blocks(wid), body, 0)


@functools.partial(
    pl.kernel,
    mesh=_mesh,
    compiler_params=pltpu.CompilerParams(use_tc_tiling_on_sc=False),
    out_type=jax.ShapeDtypeStruct((_NC, _NP, _H), jnp.float32),
    scratch_types=(
        pltpu.VMEM((_KB, _W), jnp.int32),
        pltpu.VMEM((_KB, _W), jnp.int32),
        tuple(pltpu.VMEM((_W, _H), jnp.float32) for _ in range(_NR)),
        tuple(pltpu.SemaphoreType.DMA for _ in range(2 * _NR)),
        pltpu.VMEM_SHARED((_NP, _H), jnp.float32),
    ),
)
def _seg1(e3_hbm, table_hbm, zacc_hbm,
          out_hbm,
          srcb, dstb, bufs, sems, acc_sh):
    c = lax.axis_index("c")
    s = lax.axis_index("s")
    wid = s * _NC + c
    roff = s * _CHUNK
    pltpu.sync_copy(zacc_hbm, acc_sh.at[pl.ds(roff, _CHUNK)])
    plsc.subcore_barrier()
    _ring_pass(e3_hbm, srcb, dstb, wid, table_hbm, acc_sh,
               bufs, sems[:_NR], sems[_NR:])
    plsc.subcore_barrier()
    pltpu.sync_copy(acc_sh.at[pl.ds(roff, _CHUNK)],
                    out_hbm.at[c, pl.ds(roff, _CHUNK)])


@functools.partial(
    pl.kernel,
    mesh=_mesh,
    compiler_params=pltpu.CompilerParams(use_tc_tiling_on_sc=False),
    out_type=jax.ShapeDtypeStruct((_NC, _NP, _DEGW), jnp.float32),
    scratch_types=(
        pltpu.VMEM((_KB, _W), jnp.int32),
        pltpu.VMEM((_W, _DEGW), jnp.float32),
        tuple(pltpu.SemaphoreType.DMA for _ in range(_NR)),
        pltpu.VMEM_SHARED((_NP, _DEGW), jnp.float32),
    ),
)
def _deg(e3_hbm, zdeg_hbm, ones_hbm,
         deg_hbm,
         dstb, onesv, sems, deg_sh):
    c = lax.axis_index("c")
    s = lax.axis_index("s")
    wid = s * _NC + c
    roff = s * _CHUNK
    pltpu.sync_copy(zdeg_hbm, deg_sh.at[pl.ds(roff, _CHUNK)])
    pltpu.sync_copy(ones_hbm, onesv)
    plsc.subcore_barrier()

    def body(i, carry):
        blk = wid + i * _NWK
        pltpu.sync_copy(e3_hbm.at[1, pl.ds(blk * _KB, _KB)], dstb)
        scp = [None] * _KB
        for j in range(_KB):
            if j >= _NR:
                scp[j - _NR].wait()
            scp[j] = pltpu.async_copy(onesv, deg_sh.at[dstb.at[j]],
                                      sems[j % _NR], add=True)
        for p in range(_KB - _NR, _KB):
            scp[p].wait()
        return carry

    lax.fori_loop(0, _worker_blocks(wid), body, 0)
    plsc.subcore_barrier()
    pltpu.sync_copy(deg_sh.at[pl.ds(roff, _CHUNK)],
                    deg_hbm.at[c, pl.ds(roff, _CHUNK)])


@functools.partial(
    pl.kernel,
    mesh=_mesh,
    compiler_params=pltpu.CompilerParams(use_tc_tiling_on_sc=False),
    out_type=jax.ShapeDtypeStruct((_NC, _NP, 8), jnp.float32),
    scratch_types=(
        pltpu.VMEM((_KB, _W), jnp.int32),
        pltpu.VMEM((_KB, _W), jnp.int32),
        tuple(pltpu.VMEM((_W, 8), jnp.float32) for _ in range(_NR)),
        tuple(pltpu.SemaphoreType.DMA for _ in range(2 * _NR)),
        pltpu.VMEM_SHARED((_NP, 8), jnp.float32),
        pltpu.VMEM_SHARED((_NP, 8), jnp.float32),
    ),
)
def _seg2(e3_hbm, table_hbm, zacc_hbm,
          out_hbm,
          srcb, dstb, bufs, sems, acc_sh, table_sh):
    c = lax.axis_index("c")
    s = lax.axis_index("s")
    wid = s * _NC + c
    roff = s * _CHUNK
    pltpu.sync_copy(zacc_hbm, acc_sh.at[pl.ds(roff, _CHUNK)])
    pltpu.sync_copy(table_hbm.at[pl.ds(roff, _CHUNK)],
                    table_sh.at[pl.ds(roff, _CHUNK)])
    plsc.subcore_barrier()
    _ring_pass(e3_hbm, srcb, dstb, wid, table_sh, acc_sh,
               bufs, sems[:_NR], sems[_NR:])
    plsc.subcore_barrier()
    pltpu.sync_copy(acc_sh.at[pl.ds(roff, _CHUNK)],
                    out_hbm.at[c, pl.ds(roff, _CHUNK)])


_NF32 = _NP * _H // 128
_NF8 = _NP * 8 // 128

_PROJB = 2000
_MIDB = _NF32 // 2
_FINB = _NF8


def _proj1_body(x_ref, w_ref, xn_ref, xs_ref):
    xw = jnp.dot(x_ref[...], w_ref[...], preferred_element_type=jnp.float32)
    xn_ref[...] = xw[:, :_H]
    xs_ref[...] = xw[:, _H:]


_proj1 = pl.pallas_call(
    _proj1_body,
    grid=(_N // _PROJB,),
    in_specs=[
        pl.BlockSpec((_PROJB, _D), lambda i: (i, 0)),
        pl.BlockSpec((_D, 2 * _H), lambda i: (0, 0)),
    ],
    out_specs=[
        pl.BlockSpec((_PROJB, _H), lambda i: (i, 0)),
        pl.BlockSpec((_PROJB, _H), lambda i: (i, 0)),
    ],
    out_shape=[
        jax.ShapeDtypeStruct((_NP, _H), jnp.float32),
        jax.ShapeDtypeStruct((_NP, _H), jnp.float32),
    ],
)


def _mid_body(s1_ref, deg_ref, xs_ref, b1_ref, w2bd_ref, csel_ref, hh_ref):
    ssum = s1_ref[0] + s1_ref[1]
    deg = deg_ref[0] + deg_ref[1]
    rec = 1.0 / jnp.maximum(deg, 1.0)
    h = jnp.maximum(ssum * rec + b1_ref[...] + xs_ref[...], 0.0)
    hh_ref[...] = (
        jnp.dot(h, w2bd_ref[...], preferred_element_type=jnp.float32)
        + jnp.dot(rec, csel_ref[...], preferred_element_type=jnp.float32))


_mid = pl.pallas_call(
    _mid_body,
    grid=(_NF32 // _MIDB,),
    in_specs=[
        pl.BlockSpec((_NC, _MIDB, 128), lambda i: (0, i, 0)),
        pl.BlockSpec((_NC, _MIDB, 128), lambda i: (0, i, 0)),
        pl.BlockSpec((_MIDB, 128), lambda i: (i, 0)),
        pl.BlockSpec((1, 128), lambda i: (0, 0)),
        pl.BlockSpec((128, 32), lambda i: (0, 0)),
        pl.BlockSpec((128, 32), lambda i: (0, 0)),
    ],
    out_specs=pl.BlockSpec((_MIDB, 32), lambda i: (i, 0)),
    out_shape=jax.ShapeDtypeStruct((_NF32, 32), jnp.float32),
)


def _final_body(s2_ref, hh_ref, b2_ref, phs_ref, prec_ref, pswap_ref,
                ppack_ref, out_ref):
    s2 = s2_ref[0] + s2_ref[1]
    hh = hh_ref[...]
    hs = jnp.dot(hh, phs_ref[...], preferred_element_type=jnp.float32)
    rb = jnp.dot(hh, prec_ref[...], preferred_element_type=jnp.float32)
    o = s2 * rb + b2_ref[...] + hs
    osw = jnp.dot(o, pswap_ref[...], preferred_element_type=jnp.float32)
    m = jnp.maximum(o, osw)
    e = jnp.exp(o - m)
    esw = jnp.dot(e, pswap_ref[...], preferred_element_type=jnp.float32)
    res = o - (m + jnp.log(e + esw))
    out_ref[...] = jnp.dot(res, ppack_ref[...],
                           preferred_element_type=jnp.float32)


_final = pl.pallas_call(
    _final_body,
    grid=(_NF8 // _FINB,),
    in_specs=[
        pl.BlockSpec((_NC, _FINB, 128), lambda i: (0, i, 0)),
        pl.BlockSpec((_FINB, 128), lambda i: (i, 0)),
        pl.BlockSpec((1, 128), lambda i: (0, 0)),
        pl.BlockSpec((128, 128), lambda i: (0, 0)),
        pl.BlockSpec((128, 128), lambda i: (0, 0)),
        pl.BlockSpec((128, 128), lambda i: (0, 0)),
        pl.BlockSpec((128, 32), lambda i: (0, 0)),
    ],
    out_specs=pl.BlockSpec((_FINB, 32), lambda i: (i, 0)),
    out_shape=jax.ShapeDtypeStruct((_NF8, 32), jnp.float32),
)


def kernel(x, edge_index, W1_neigh, W1_self, b1, W2_neigh, W2_self, b2):
    e3 = edge_index.reshape(2, _E // _W, _W)
    w1 = jnp.concatenate([W1_neigh, W1_self], axis=1)
    w2 = jnp.concatenate([W2_neigh, W2_self], axis=1)

    w2bd = jnp.kron(jnp.asarray(np.eye(4, dtype=np.float32)),
                    jnp.pad(w2, ((0, 0), (0, 4))))
    csel_np = np.zeros((128, 32), np.float32)
    phs_np = np.zeros((128, 128), np.float32)
    prec_np = np.zeros((128, 128), np.float32)
    pswap_np = np.zeros((128, 128), np.float32)
    ppack_np = np.zeros((128, 32), np.float32)
    for g in range(4):
        csel_np[32 * g, 8 * g + 4] = 1.0
        csel_np[32 * g, 8 * g + 5] = 1.0
    for g in range(16):
        phs_np[8 * g + 2, 8 * g] = 1.0
        phs_np[8 * g + 3, 8 * g + 1] = 1.0
        prec_np[8 * g + 4, 8 * g] = 1.0
        prec_np[8 * g + 4, 8 * g + 1] = 1.0
        pswap_np[8 * g, 8 * g + 1] = 1.0
        pswap_np[8 * g + 1, 8 * g] = 1.0
        ppack_np[8 * g, 2 * g] = 1.0
        ppack_np[8 * g + 1, 2 * g + 1] = 1.0
    csel = jnp.asarray(csel_np)
    phs = jnp.asarray(phs_np)
    prec = jnp.asarray(prec_np)
    pswap = jnp.asarray(pswap_np)
    ppack = jnp.asarray(ppack_np)
    b2t = jnp.tile(jnp.pad(b2, (0, 6)), 16).reshape(1, 128)
    b1t = jnp.tile(b1, 4).reshape(1, 128)

    zdeg = jnp.zeros((_CHUNK, _DEGW), jnp.float32)
    ones = jnp.ones((_W, _DEGW), jnp.float32)
    degp = _deg(e3, zdeg, ones)

    xn, xs = _proj1(x, w1)

    zacc = jnp.zeros((_CHUNK, _H), jnp.float32)
    s1p = _seg1(e3, xn, zacc)

    s1f = s1p.reshape(_NC, _NF32, 128)
    degf = degp.reshape(_NC, _NF32, 128)
    xsf = xs.reshape(_NF32, 128)
    hhf = _mid(s1f, degf, xsf, b1t, w2bd, csel)
    hh = hhf.reshape(_NP, 8)

    zacc2 = jnp.zeros((_CHUNK, 8), jnp.float32) + 0.0 * degp[0, 0, 0]
    s2p = _seg2(e3, hh, zacc2)

    s2f = s2p.reshape(_NC, _NF8, 128)
    outf = _final(s2f, hhf.reshape(_NF8, 128), b2t, phs, prec, pswap, ppack)
    return outf.reshape(_NP, _C)[:_N]


# speedup vs baseline: 1.1537x; 1.1537x over previous
import functools

import jax
import jax.numpy as jnp
import numpy as np
from jax import lax
from jax.experimental import pallas as pl
from jax.experimental.pallas import tpu as pltpu
from jax.experimental.pallas import tpu_sc as plsc

_N = 50000
_E = 800000
_D = 100
_H = 32
_C = 2

_NC = 2
_NS = 16
_NWK = _NC * _NS
_W = 128
_KB = 25
_NBLKE = _E // (_W * _KB)
_NP = 50048
_CHUNK = _NP // _NS
_DEGW = 32
_NR = 5
_LAG = 3

_mesh = plsc.VectorSubcoreMesh(core_axis_name="c", subcore_axis_name="s")


def _worker_blocks(wid):
    return (_NBLKE - wid + _NWK - 1) // _NWK


def _ring_pass(e3_hbm, srcb, dstb, wid, table_ref, acc_sh, bufs, gsems, ssems):
    def body(i, carry):
        blk = wid + i * _NWK
        pltpu.sync_copy(e3_hbm.at[0, pl.ds(blk * _KB, _KB)], srcb)
        pltpu.sync_copy(e3_hbm.at[1, pl.ds(blk * _KB, _KB)], dstb)
        gcp = [None] * _KB
        scp = [None] * _KB
        for j in range(_KB):
            b = j % _NR
            if j >= _NR:
                scp[j - _NR].wait()
            gcp[j] = pltpu.async_copy(table_ref.at[srcb.at[j]], bufs[b],
                                      gsems[b])
            if j >= _LAG:
                p = j - _LAG
                gcp[p].wait()
                scp[p] = pltpu.async_copy(bufs[p % _NR],
                                          acc_sh.at[dstb.at[p]],
                                          ssems[p % _NR], add=True)
        for p in range(_KB - _LAG, _KB):
            gcp[p].wait()
            scp[p] = pltpu.async_copy(bufs[p % _NR], acc_sh.at[dstb.at[p]],
                                      ssems[p % _NR], add=True)
        for p in range(_KB - _NR, _KB):
            scp[p].wait()
        return carry

    lax.fori_loop(0, _worker_blocks(wid), body, 0)


@functools.partial(
    pl.kernel,
    mesh=_mesh,
    compiler_params=pltpu.CompilerParams(use_tc_tiling_on_sc=False),
    out_type=jax.ShapeDtypeStruct((_NC, _NP, _H), jnp.float32),
    scratch_types=(
        pltpu.VMEM((_KB, _W), jnp.int32),
        pltpu.VMEM((_KB, _W), jnp.int32),
        tuple(pltpu.VMEM((_W, _H), jnp.float32) for _ in range(_NR)),
        tuple(pltpu.SemaphoreType.DMA for _ in range(2 * _NR)),
        pltpu.VMEM_SHARED((_NP, _H), jnp.float32),
    ),
)
def _seg1(e3_hbm, table_hbm, zacc_hbm,
          out_hbm,
          srcb, dstb, bufs, sems, acc_sh):
    c = lax.axis_index("c")
    s = lax.axis_index("s")
    wid = s * _NC + c
    roff = s * _CHUNK
    pltpu.sync_copy(zacc_hbm, acc_sh.at[pl.ds(roff, _CHUNK)])
    plsc.subcore_barrier()
    _ring_pass(e3_hbm, srcb, dstb, wid, table_hbm, acc_sh,
               bufs, sems[:_NR], sems[_NR:])
    plsc.subcore_barrier()
    pltpu.sync_copy(acc_sh.at[pl.ds(roff, _CHUNK)],
                    out_hbm.at[c, pl.ds(roff, _CHUNK)])


@functools.partial(
    pl.kernel,
    mesh=_mesh,
    compiler_params=pltpu.CompilerParams(use_tc_tiling_on_sc=False),
    out_type=jax.ShapeDtypeStruct((_NC, _NP, _DEGW), jnp.float32),
    scratch_types=(
        pltpu.VMEM((_KB, _W), jnp.int32),
        pltpu.VMEM((_W, _DEGW), jnp.float32),
        tuple(pltpu.SemaphoreType.DMA for _ in range(_NR)),
        pltpu.VMEM_SHARED((_NP, _DEGW), jnp.float32),
    ),
)
def _deg(e3_hbm, zdeg_hbm, ones_hbm,
         deg_hbm,
         dstb, onesv, sems, deg_sh):
    c = lax.axis_index("c")
    s = lax.axis_index("s")
    wid = s * _NC + c
    roff = s * _CHUNK
    pltpu.sync_copy(zdeg_hbm, deg_sh.at[pl.ds(roff, _CHUNK)])
    pltpu.sync_copy(ones_hbm, onesv)
    plsc.subcore_barrier()

    def body(i, carry):
        blk = wid + i * _NWK
        pltpu.sync_copy(e3_hbm.at[1, pl.ds(blk * _KB, _KB)], dstb)
        scp = [None] * _KB
        for j in range(_KB):
            if j >= _NR:
                scp[j - _NR].wait()
            scp[j] = pltpu.async_copy(onesv, deg_sh.at[dstb.at[j]],
                                      sems[j % _NR], add=True)
        for p in range(_KB - _NR, _KB):
            scp[p].wait()
        return carry

    lax.fori_loop(0, _worker_blocks(wid), body, 0)
    plsc.subcore_barrier()
    pltpu.sync_copy(deg_sh.at[pl.ds(roff, _CHUNK)],
                    deg_hbm.at[c, pl.ds(roff, _CHUNK)])


@functools.partial(
    pl.kernel,
    mesh=_mesh,
    compiler_params=pltpu.CompilerParams(use_tc_tiling_on_sc=False),
    out_type=jax.ShapeDtypeStruct((_NC, _NP, 8), jnp.float32),
    scratch_types=(
        pltpu.VMEM((_KB, _W), jnp.int32),
        pltpu.VMEM((_KB, _W), jnp.int32),
        tuple(pltpu.VMEM((_W, 8), jnp.float32) for _ in range(_NR)),
        tuple(pltpu.SemaphoreType.DMA for _ in range(2 * _NR)),
        pltpu.VMEM_SHARED((_NP, 8), jnp.float32),
        pltpu.VMEM_SHARED((_NP, 8), jnp.float32),
    ),
)
def _seg2(e3_hbm, table_hbm, zacc_hbm,
          out_hbm,
          srcb, dstb, bufs, sems, acc_sh, table_sh):
    c = lax.axis_index("c")
    s = lax.axis_index("s")
    wid = s * _NC + c
    roff = s * _CHUNK
    pltpu.sync_copy(zacc_hbm, acc_sh.at[pl.ds(roff, _CHUNK)])
    pltpu.sync_copy(table_hbm.at[pl.ds(roff, _CHUNK)],
                    table_sh.at[pl.ds(roff, _CHUNK)])
    plsc.subcore_barrier()
    _ring_pass(e3_hbm, srcb, dstb, wid, table_sh, acc_sh,
               bufs, sems[:_NR], sems[_NR:])
    plsc.subcore_barrier()
    pltpu.sync_copy(acc_sh.at[pl.ds(roff, _CHUNK)],
                    out_hbm.at[c, pl.ds(roff, _CHUNK)])


_NF32 = _NP * _H // 128
_NF8 = _NP * 8 // 128

_PROJB = 2000
_MIDB = _NF32 // 2
_FINB = _NF8


def _proj1_body(x_ref, w_ref, xn_ref, xs_ref):
    xw = jnp.dot(x_ref[...], w_ref[...], preferred_element_type=jnp.float32)
    xn_ref[...] = xw[:, :_H]
    xs_ref[...] = xw[:, _H:]


_proj1 = pl.pallas_call(
    _proj1_body,
    grid=(_N // _PROJB,),
    in_specs=[
        pl.BlockSpec((_PROJB, _D), lambda i: (i, 0)),
        pl.BlockSpec((_D, 2 * _H), lambda i: (0, 0)),
    ],
    out_specs=[
        pl.BlockSpec((_PROJB, _H), lambda i: (i, 0)),
        pl.BlockSpec((_PROJB, _H), lambda i: (i, 0)),
    ],
    out_shape=[
        jax.ShapeDtypeStruct((_NP, _H), jnp.float32),
        jax.ShapeDtypeStruct((_NP, _H), jnp.float32),
    ],
)


def _mid_body(s1_ref, deg_ref, xs_ref, b1_ref, w2bd_ref, csel_ref, hh_ref):
    ssum = s1_ref[0] + s1_ref[1]
    deg = deg_ref[0] + deg_ref[1]
    rec = 1.0 / jnp.maximum(deg, 1.0)
    h = jnp.maximum(ssum * rec + b1_ref[...] + xs_ref[...], 0.0)
    hh_ref[...] = (
        jnp.dot(h, w2bd_ref[...], preferred_element_type=jnp.float32)
        + jnp.dot(rec, csel_ref[...], preferred_element_type=jnp.float32))


_mid = pl.pallas_call(
    _mid_body,
    grid=(_NF32 // _MIDB,),
    in_specs=[
        pl.BlockSpec((_NC, _MIDB, 128), lambda i: (0, i, 0)),
        pl.BlockSpec((_NC, _MIDB, 128), lambda i: (0, i, 0)),
        pl.BlockSpec((_MIDB, 128), lambda i: (i, 0)),
        pl.BlockSpec((1, 128), lambda i: (0, 0)),
        pl.BlockSpec((128, 32), lambda i: (0, 0)),
        pl.BlockSpec((128, 32), lambda i: (0, 0)),
    ],
    out_specs=pl.BlockSpec((_MIDB, 32), lambda i: (i, 0)),
    out_shape=jax.ShapeDtypeStruct((_NF32, 32), jnp.float32),
)


def _final_body(s2_ref, hh_ref, b2_ref, phs_ref, prec_ref, pswap_ref,
                ppack_ref, out_ref):
    s2 = s2_ref[0] + s2_ref[1]
    hh = hh_ref[...]
    hs = jnp.dot(hh, phs_ref[...], preferred_element_type=jnp.float32)
    rb = jnp.dot(hh, prec_ref[...], preferred_element_type=jnp.float32)
    o = s2 * rb + b2_ref[...] + hs
    osw = jnp.dot(o, pswap_ref[...], preferred_element_type=jnp.float32)
    m = jnp.maximum(o, osw)
    e = jnp.exp(o - m)
    esw = jnp.dot(e, pswap_ref[...], preferred_element_type=jnp.float32)
    res = o - (m + jnp.log(e + esw))
    out_ref[...] = jnp.dot(res, ppack_ref[...],
                           preferred_element_type=jnp.float32)


_final = pl.pallas_call(
    _final_body,
    grid=(_NF8 // _FINB,),
    in_specs=[
        pl.BlockSpec((_NC, _FINB, 128), lambda i: (0, i, 0)),
        pl.BlockSpec((_FINB, 128), lambda i: (i, 0)),
        pl.BlockSpec((1, 128), lambda i: (0, 0)),
        pl.BlockSpec((128, 128), lambda i: (0, 0)),
        pl.BlockSpec((128, 128), lambda i: (0, 0)),
        pl.BlockSpec((128, 128), lambda i: (0, 0)),
        pl.BlockSpec((128, 32), lambda i: (0, 0)),
    ],
    out_specs=pl.BlockSpec((_FINB, 32), lambda i: (i, 0)),
    out_shape=jax.ShapeDtypeStruct((_NF8, 32), jnp.float32),
)


def kernel(x, edge_index, W1_neigh, W1_self, b1, W2_neigh, W2_self, b2):
    e3 = edge_index.reshape(2, _E // _W, _W)
    w1 = jnp.concatenate([W1_neigh, W1_self], axis=1)
    w2 = jnp.concatenate([W2_neigh, W2_self], axis=1)

    w2bd = jnp.kron(jnp.asarray(np.eye(4, dtype=np.float32)),
                    jnp.pad(w2, ((0, 0), (0, 4))))
    csel_np = np.zeros((128, 32), np.float32)
    phs_np = np.zeros((128, 128), np.float32)
    prec_np = np.zeros((128, 128), np.float32)
    pswap_np = np.zeros((128, 128), np.float32)
    ppack_np = np.zeros((128, 32), np.float32)
    for g in range(4):
        csel_np[32 * g, 8 * g + 4] = 1.0
        csel_np[32 * g, 8 * g + 5] = 1.0
    for g in range(16):
        phs_np[8 * g + 2, 8 * g] = 1.0
        phs_np[8 * g + 3, 8 * g + 1] = 1.0
        prec_np[8 * g + 4, 8 * g] = 1.0
        prec_np[8 * g + 4, 8 * g + 1] = 1.0
        pswap_np[8 * g, 8 * g + 1] = 1.0
        pswap_np[8 * g + 1, 8 * g] = 1.0
        ppack_np[8 * g, 2 * g] = 1.0
        ppack_np[8 * g + 1, 2 * g + 1] = 1.0
    csel = jnp.asarray(csel_np)
    phs = jnp.asarray(phs_np)
    prec = jnp.asarray(prec_np)
    pswap = jnp.asarray(pswap_np)
    ppack = jnp.asarray(ppack_np)
    b2t = jnp.tile(jnp.pad(b2, (0, 6)), 16).reshape(1, 128)
    b1t = jnp.tile(b1, 4).reshape(1, 128)

    zdeg = jnp.zeros((_CHUNK, _DEGW), jnp.float32)
    ones = jnp.ones((_W, _DEGW), jnp.float32)
    degp = _deg(e3, zdeg, ones)

    xn, xs = _proj1(x, w1)

    zacc = jnp.zeros((_CHUNK, _H), jnp.float32) + 0.0 * degp[0, 0, 0]
    s1p = _seg1(e3, xn, zacc)

    s1f = s1p.reshape(_NC, _NF32, 128)
    degf = degp.reshape(_NC, _NF32, 128)
    xsf = xs.reshape(_NF32, 128)
    hhf = _mid(s1f, degf, xsf, b1t, w2bd, csel)
    hh = hhf.reshape(_NP, 8)

    zacc2 = jnp.zeros((_CHUNK, 8), jnp.float32)
    s2p = _seg2(e3, hh, zacc2)

    s2f = s2p.reshape(_NC, _NF8, 128)
    outf = _final(s2f, hhf.reshape(_NF8, 128), b2t, phs, prec, pswap, ppack)
    return outf.reshape(_NP, _C)[:_N]
